# baseline (speedup 1.0000x reference)
import functools

import jax
import jax.numpy as jnp
from jax import lax
from jax.experimental import pallas as pl
from jax.experimental.pallas import tpu as pltpu
from jax.experimental.pallas import tpu_sc as plsc

B = 4096
S = 200
D = 64
NSPAN = S // 2
N = B * NSPAN

NC = 2
NS = 16
NW = NC * NS
ROWS_PER_W = N // NW
CHUNK = 256
NCHUNK = ROWS_PER_W // CHUNK

_mesh = plsc.VectorSubcoreMesh(core_axis_name="c", subcore_axis_name="s")


@functools.partial(
    pl.kernel,
    mesh=_mesh,
    out_type=(
        jax.ShapeDtypeStruct((N, D), jnp.float32),
        jax.ShapeDtypeStruct((N, D), jnp.float32),
        jax.ShapeDtypeStruct((N,), jnp.int32),
    ),
    scratch_types=[
        pltpu.VMEM((CHUNK, 2, D), jnp.float32),
        pltpu.VMEM((CHUNK,), jnp.int32),
    ],
)
def _deinterleave(x_hbm, start_hbm, end_hbm, len_hbm, buf, len_buf):
    wid = lax.axis_index("s") * NC + lax.axis_index("c")
    base_w = wid * ROWS_PER_W

    two = jnp.full((16,), 2, jnp.int32)
    for j in range(CHUNK // 16):
        len_buf[pl.ds(j * 16, 16)] = two

    def body(i, carry):
        base = base_w + i * CHUNK
        pltpu.sync_copy(x_hbm.at[pl.ds(base, CHUNK)], buf)
        pltpu.sync_copy(buf.at[:, 0], start_hbm.at[pl.ds(base, CHUNK)])
        pltpu.sync_copy(buf.at[:, 1], end_hbm.at[pl.ds(base, CHUNK)])
        pltpu.sync_copy(len_buf, len_hbm.at[pl.ds(base, CHUNK)])
        return carry

    lax.fori_loop(0, NCHUNK, body, 0)


def kernel(input):
    x3 = input.reshape(N, 2, D)
    s, e, ln = _deinterleave(x3)
    return (
        s.reshape(B, NSPAN, D),
        e.reshape(B, NSPAN, D),
        ln.reshape(B, NSPAN),
    )


# speedup vs baseline: 1.0246x; 1.0246x over previous
import functools

import jax
import jax.numpy as jnp
from jax import lax
from jax.experimental import pallas as pl
from jax.experimental.pallas import tpu as pltpu
from jax.experimental.pallas import tpu_sc as plsc

B = 4096
S = 200
D = 64
NSPAN = S // 2
N = B * NSPAN

NC = 2
NS = 16
NW = NC * NS
ROWS_PER_W = N // NW
CHUNK = 200
NCHUNK = ROWS_PER_W // CHUNK

_mesh = plsc.VectorSubcoreMesh(core_axis_name="c", subcore_axis_name="s")


@functools.partial(
    pl.kernel,
    mesh=_mesh,
    out_type=(
        jax.ShapeDtypeStruct((N, D), jnp.float32),
        jax.ShapeDtypeStruct((N, D), jnp.float32),
        jax.ShapeDtypeStruct((N,), jnp.int32),
    ),
    scratch_types=[
        pltpu.VMEM((CHUNK, 2, D), jnp.float32),
        pltpu.VMEM((CHUNK, 2, D), jnp.float32),
        pltpu.VMEM((CHUNK,), jnp.int32),
        pltpu.SemaphoreType.DMA,
        pltpu.SemaphoreType.DMA,
    ],
)
def _deinterleave(x_hbm, start_hbm, end_hbm, len_hbm, buf0, buf1, len_buf,
                  sem_r, sem_w):
    wid = lax.axis_index("s") * NC + lax.axis_index("c")
    base_w = wid * ROWS_PER_W

    two = jnp.full((16,), 2, jnp.int32)
    for j in range(CHUNK // 16 + 1):
        off = min(j * 16, CHUNK - 16)
        len_buf[pl.ds(off, 16)] = two

    bufs = (buf0, buf1)

    pltpu.async_copy(x_hbm.at[pl.ds(base_w, CHUNK)], buf0, sem_r)

    def body(g, carry):
        for b in range(2):
            i = 2 * g + b
            cur = bufs[b]
            nxt = bufs[1 - b]
            base = base_w + i * CHUNK
            nbase = jnp.where(i + 1 < NCHUNK, base + CHUNK, base_w)
            pltpu.make_async_copy(x_hbm.at[pl.ds(base, CHUNK)], cur, sem_r).wait()
            pltpu.async_copy(x_hbm.at[pl.ds(nbase, CHUNK)], nxt, sem_r)
            pltpu.async_copy(cur.at[:, 0], start_hbm.at[pl.ds(base, CHUNK)], sem_w)
            pltpu.async_copy(cur.at[:, 1], end_hbm.at[pl.ds(base, CHUNK)], sem_w)
            pltpu.async_copy(len_buf, len_hbm.at[pl.ds(base, CHUNK)], sem_w)
            pltpu.make_async_copy(cur.at[:, 0], start_hbm.at[pl.ds(base, CHUNK)], sem_w).wait()
            pltpu.make_async_copy(cur.at[:, 1], end_hbm.at[pl.ds(base, CHUNK)], sem_w).wait()
            pltpu.make_async_copy(len_buf, len_hbm.at[pl.ds(base, CHUNK)], sem_w).wait()
        return carry

    lax.fori_loop(0, NCHUNK // 2, body, 0)
    pltpu.make_async_copy(x_hbm.at[pl.ds(base_w, CHUNK)], bufs[0], sem_r).wait()


def kernel(input):
    x3 = input.reshape(N, 2, D)
    s, e, ln = _deinterleave(x3)
    return (
        s.reshape(B, NSPAN, D),
        e.reshape(B, NSPAN, D),
        ln.reshape(B, NSPAN),
    )


# speedup vs baseline: 1.1364x; 1.1091x over previous
import functools

import jax
import jax.numpy as jnp
from jax import lax
from jax.experimental import pallas as pl
from jax.experimental.pallas import tpu as pltpu
from jax.experimental.pallas import tpu_sc as plsc

B = 4096
S = 200
D = 64
NSPAN = S // 2

NC = 2
NS = 16
NW = NC * NS
BATCH_PER_W = B // NW
BCH = 2
NCHUNK = BATCH_PER_W // BCH

_mesh = plsc.VectorSubcoreMesh(core_axis_name="c", subcore_axis_name="s")


@functools.partial(
    pl.kernel,
    mesh=_mesh,
    out_type=(
        jax.ShapeDtypeStruct((B, NSPAN, D), jnp.float32),
        jax.ShapeDtypeStruct((B, NSPAN, D), jnp.float32),
        jax.ShapeDtypeStruct((B, NSPAN), jnp.int32),
    ),
    scratch_types=[
        pltpu.VMEM((BCH, S, D), jnp.float32),
        pltpu.VMEM((BCH, S, D), jnp.float32),
        pltpu.VMEM((BCH, NSPAN), jnp.int32),
        pltpu.SemaphoreType.DMA,
        pltpu.SemaphoreType.DMA,
    ],
)
def _deinterleave(x_hbm, start_hbm, end_hbm, len_hbm, buf0, buf1, len_buf,
                  sem_r, sem_w):
    wid = lax.axis_index("s") * NC + lax.axis_index("c")
    base_w = wid * BATCH_PER_W

    two = jnp.full((16,), 2, jnp.int32)
    for r in range(BCH):
        for j in range(NSPAN // 16 + 1):
            off = min(j * 16, NSPAN - 16)
            len_buf[r, pl.ds(off, 16)] = two

    bufs = (buf0, buf1)
    views = (buf0.reshape(BCH, NSPAN, 2, D), buf1.reshape(BCH, NSPAN, 2, D))

    pltpu.async_copy(x_hbm.at[pl.ds(base_w, BCH)], buf0, sem_r)

    def body(g, carry):
        for b in range(2):
            i = 2 * g + b
            cur, cur_v = bufs[b], views[b]
            nxt = bufs[1 - b]
            base = base_w + i * BCH
            nbase = jnp.where(i + 1 < NCHUNK, base + BCH, base_w)
            pltpu.make_async_copy(x_hbm.at[pl.ds(base, BCH)], cur, sem_r).wait()
            pltpu.async_copy(x_hbm.at[pl.ds(nbase, BCH)], nxt, sem_r)
            pltpu.async_copy(cur_v.at[:, :, 0], start_hbm.at[pl.ds(base, BCH)], sem_w)
            pltpu.async_copy(cur_v.at[:, :, 1], end_hbm.at[pl.ds(base, BCH)], sem_w)
            pltpu.async_copy(len_buf, len_hbm.at[pl.ds(base, BCH)], sem_w)
            pltpu.make_async_copy(cur_v.at[:, :, 0], start_hbm.at[pl.ds(base, BCH)], sem_w).wait()
            pltpu.make_async_copy(cur_v.at[:, :, 1], end_hbm.at[pl.ds(base, BCH)], sem_w).wait()
            pltpu.make_async_copy(len_buf, len_hbm.at[pl.ds(base, BCH)], sem_w).wait()
        return carry

    lax.fori_loop(0, NCHUNK // 2, body, 0)
    pltpu.make_async_copy(x_hbm.at[pl.ds(base_w, BCH)], bufs[0], sem_r).wait()


def kernel(input):
    return _deinterleave(input)
